# baseline (speedup 1.0000x reference)
import functools

import jax
import jax.numpy as jnp
from jax import lax
from jax.experimental import pallas as pl
from jax.experimental.pallas import tpu as pltpu
from jax.experimental.pallas import tpu_sc as plsc

F32 = jnp.float32

_N1 = 5000
_N2 = 5000
_N = _N1 + _N2
_E = 320000
_D = 128

_NC = 2
_NS = 16
_NW = _NC * _NS
_CH = 256
_CHC = 128
_NCHUNK = 40
_EPT = _NCHUNK * _CH
_EPAD = _EPT * _NW
_NPAD = _N + 112
_RPT = _NPAD // _NS


def _dot_t(a, w):
    return lax.dot_general(a, w, (((1,), (1,)), ((), ())),
                           precision=lax.Precision.HIGHEST,
                           preferred_element_type=F32)


_BR = 1000
_NB = _N // _BR

_full = pl.BlockSpec((1, _D), lambda i: (0, 0))
_wfull = pl.BlockSpec((_D, _D), lambda i: (0, 0))
_rows = pl.BlockSpec((_BR, _D), lambda i: (i, 0))
_acc_bs = pl.BlockSpec((2, _BR, _D), lambda i: (0, i, 0))
_rcp_bs = pl.BlockSpec((_BR, 1), lambda i: (i, 0))


def _proj(x, ws, bs, wd, bd):
    def body(x_ref, ws_ref, bs_ref, wd_ref, bd_ref, o_ref):
        i = pl.program_id(0)
        first = i < (_N1 // _BR)
        w = jnp.where(first, ws_ref[...], wd_ref[...])
        bias = jnp.where(first, bs_ref[...], bd_ref[...])
        o_ref[...] = _dot_t(x_ref[...], w) + bias

    return pl.pallas_call(
        body,
        grid=(_NB,),
        in_specs=[_rows, _wfull, _full, _wfull, _full],
        out_specs=_rows,
        out_shape=jax.ShapeDtypeStruct((_N, _D), F32))(x, ws, bs, wd, bd)


def _cnt_recip(cnt):
    def body(cnt_ref, o_ref):
        s = cnt_ref[0, :_N, 0:1] + cnt_ref[1, :_N, 0:1]
        o_ref[...] = 1.0 / jnp.maximum(s, 1.0)

    return pl.pallas_call(
        body, out_shape=jax.ShapeDtypeStruct((_N, 1), F32))(cnt)


def _sage_lin(acc, recip, h, wl, bl, wr, with_stats):
    def body(acc_ref, rcp_ref, h_ref, wl_ref, bl_ref, wr_ref, *out_refs):
        o_ref = out_refs[0]
        a = acc_ref[0] + acc_ref[1]
        mean = a * rcp_ref[...]
        hp = (_dot_t(mean, wl_ref[...]) + bl_ref[...]
              + _dot_t(h_ref[...], wr_ref[...]))
        o_ref[...] = hp
        if with_stats:
            s1_ref, s2_ref = out_refs[1], out_refs[2]

            @pl.when(pl.program_id(0) == 0)
            def _():
                s1_ref[...] = jnp.zeros_like(s1_ref)
                s2_ref[...] = jnp.zeros_like(s2_ref)

            s1_ref[...] += jnp.sum(hp, axis=0, keepdims=True)
            s2_ref[...] += jnp.sum(hp * hp, axis=0, keepdims=True)

    out_shape = [jax.ShapeDtypeStruct((_N, _D), F32)]
    out_specs = [_rows]
    if with_stats:
        out_shape += [jax.ShapeDtypeStruct((1, _D), F32)] * 2
        out_specs += [_full, _full]
    return pl.pallas_call(
        body,
        grid=(_NB,),
        in_specs=[_acc_bs, _rcp_bs, _rows, _wfull, _full, _wfull],
        out_specs=out_specs,
        out_shape=out_shape)(acc, recip, h, wl, bl, wr)


def _bn_relu(hp, s1, s2, g, b):
    def body(hp_ref, s1_ref, s2_ref, g_ref, b_ref, o_ref):
        inv_n = 1.0 / _N
        mu = s1_ref[...] * inv_n
        var = s2_ref[...] * inv_n - mu * mu
        hn = (hp_ref[...] - mu) * lax.rsqrt(var + 1e-5) * g_ref[...] + b_ref[...]
        o_ref[...] = jnp.maximum(hn, 0.0)

    return pl.pallas_call(
        body,
        grid=(_NB,),
        in_specs=[_rows, _full, _full, _full, _full],
        out_specs=_rows,
        out_shape=jax.ShapeDtypeStruct((_N, _D), F32))(hp, s1, s2, g, b)


def _sc_mesh():
    return plsc.VectorSubcoreMesh(core_axis_name="c", subcore_axis_name="s")


_BB = 256
_SPANS = [(off, min(_BB, _RPT - off)) for off in range(0, _RPT, _BB)]


def _zero_acc(zrows_hbm, rows, acc_sh, row0):
    pltpu.sync_copy(zrows_hbm, rows)
    for off, sz in _SPANS:
        pltpu.sync_copy(rows.at[pl.ds(0, sz)],
                        acc_sh.at[pl.ds(row0 + off, sz)])


def _dump_acc(acc_sh, rows, out_hbm, row0, c):
    for off, sz in _SPANS:
        pltpu.sync_copy(acc_sh.at[pl.ds(row0 + off, sz)],
                        rows.at[pl.ds(0, sz)])
        pltpu.sync_copy(rows.at[pl.ds(0, sz)],
                        out_hbm.at[pl.ds(c * _NPAD + row0 + off, sz)])


def _make_seg_sum():
    @functools.partial(
        pl.kernel,
        mesh=_sc_mesh(),
        out_type=jax.ShapeDtypeStruct((2 * _NPAD, _D), F32),
        scratch_types=[
            pltpu.VMEM_SHARED((_NPAD, _D), F32),
            pltpu.VMEM((_CH,), jnp.int32),
            pltpu.VMEM((_CH,), jnp.int32),
            pltpu.VMEM((_CH, _D), F32),
            pltpu.SemaphoreType.DMA,
        ],
    )
    def seg(h_hbm, src_hbm, dst_hbm, zrows_hbm, acc_out,
            acc_sh, sidx, didx, rows, sem):
        c = lax.axis_index("c")
        s = lax.axis_index("s")
        wid = s * _NC + c
        base = wid * _EPT
        row0 = s * _RPT

        _zero_acc(zrows_hbm, rows, acc_sh, row0)
        plsc.subcore_barrier()

        def chunk(i, carry):
            off = base + i * _CH
            pltpu.sync_copy(src_hbm.at[pl.ds(off, _CH)], sidx)
            pltpu.sync_copy(dst_hbm.at[pl.ds(off, _CH)], didx)
            pltpu.async_copy(h_hbm.at[sidx], rows, sem).wait()
            pltpu.sync_copy(rows, acc_sh.at[didx], add=True)
            return carry

        lax.fori_loop(0, _NCHUNK, chunk, 0)
        plsc.subcore_barrier()
        _dump_acc(acc_sh, rows, acc_out, row0, c)

    return seg


def _make_cnt():
    @functools.partial(
        pl.kernel,
        mesh=_sc_mesh(),
        out_type=jax.ShapeDtypeStruct((2 * _NPAD, _D), F32),
        scratch_types=[
            pltpu.VMEM_SHARED((_NPAD, _D), F32),
            pltpu.VMEM((_CHC,), jnp.int32),
            pltpu.VMEM((_CHC, _D), F32),
            pltpu.VMEM((_BB, _D), F32),
        ],
    )
    def cntk(dst_hbm, zrows_hbm, ones_hbm, cnt_out,
             cnt_sh, didx, ones_v, rows):
        c = lax.axis_index("c")
        s = lax.axis_index("s")
        wid = s * _NC + c
        base = wid * _EPT
        row0 = s * _RPT

        _zero_acc(zrows_hbm, rows, cnt_sh, row0)
        pltpu.sync_copy(ones_hbm, ones_v)
        plsc.subcore_barrier()

        def chunkc(i, carry):
            pltpu.sync_copy(dst_hbm.at[pl.ds(base + i * _CHC, _CHC)], didx)
            pltpu.sync_copy(ones_v, cnt_sh.at[didx], add=True)
            return carry

        lax.fori_loop(0, _EPT // _CHC, chunkc, 0)
        plsc.subcore_barrier()
        _dump_acc(cnt_sh, rows, cnt_out, row0, c)

    return cntk


_seg_sum = _make_seg_sum()
_cnt_scatter = _make_cnt()


def kernel(x, edge_index, num_nodes_type_1, num_nodes_type_2,
           W_src, b_src, W_dst, b_dst,
           Wl0, bl0, Wr0, gamma0, beta0, Wl1, bl1, Wr1):
    del num_nodes_type_1, num_nodes_type_2
    src = edge_index[0]
    dst = edge_index[1]
    npad = _EPAD - _E
    src_p = jnp.concatenate([src, jnp.zeros((npad,), jnp.int32)])
    dst_p = jnp.concatenate([dst, jnp.full((npad,), _N, jnp.int32)])
    zrows = jnp.zeros((_BB, _D), F32)
    ones_r = jnp.ones((_CHC, _D), F32)

    bs = b_src.reshape(1, _D)
    bd = b_dst.reshape(1, _D)
    bl0r = bl0.reshape(1, _D)
    bl1r = bl1.reshape(1, _D)
    g0 = gamma0.reshape(1, _D)
    be0 = beta0.reshape(1, _D)

    h = _proj(x, W_src, bs, W_dst, bd)
    cnt = _cnt_scatter(dst_p, zrows, ones_r).reshape(2, _NPAD, _D)
    recip = _cnt_recip(cnt)
    acc0 = _seg_sum(h, src_p, dst_p, zrows).reshape(2, _NPAD, _D)
    hp0, s1, s2 = _sage_lin(acc0, recip, h, Wl0, bl0r, Wr0, with_stats=True)
    h2 = _bn_relu(hp0, s1, s2, g0, be0)
    acc1 = _seg_sum(h2, src_p, dst_p, zrows).reshape(2, _NPAD, _D)
    return _sage_lin(acc1, recip, h2, Wl1, bl1r, Wr1, with_stats=False)[0]


# speedup vs baseline: 1.0134x; 1.0134x over previous
import functools

import jax
import jax.numpy as jnp
from jax import lax
from jax.experimental import pallas as pl
from jax.experimental.pallas import tpu as pltpu
from jax.experimental.pallas import tpu_sc as plsc

F32 = jnp.float32

_N1 = 5000
_N2 = 5000
_N = _N1 + _N2
_E = 320000
_D = 128

_NC = 2
_NS = 16
_NW = _NC * _NS
_CH = 128
_CHC = 128
_NCF = 100
_NCS = 60
_FAST_C = 1
_EPAD = _NS * (_NCF + _NCS) * _CH
_EPT = _EPAD // _NW
_NPAD = _N + 112
_RPT = _NPAD // _NS


def _dot_t(a, w):
    return lax.dot_general(a, w, (((1,), (1,)), ((), ())),
                           precision=lax.Precision.HIGHEST,
                           preferred_element_type=F32)


_BR = 1000
_NB = _N // _BR

_full = pl.BlockSpec((1, _D), lambda i: (0, 0))
_wfull = pl.BlockSpec((_D, _D), lambda i: (0, 0))
_rows = pl.BlockSpec((_BR, _D), lambda i: (i, 0))
_acc_bs = pl.BlockSpec((2, _BR, _D), lambda i: (0, i, 0))
_rcp_bs = pl.BlockSpec((_BR, 1), lambda i: (i, 0))


def _proj(x, ws, bs, wd, bd):
    def body(x_ref, ws_ref, bs_ref, wd_ref, bd_ref, o_ref):
        i = pl.program_id(0)
        first = i < (_N1 // _BR)
        w = jnp.where(first, ws_ref[...], wd_ref[...])
        bias = jnp.where(first, bs_ref[...], bd_ref[...])
        o_ref[...] = _dot_t(x_ref[...], w) + bias

    return pl.pallas_call(
        body,
        grid=(_NB,),
        in_specs=[_rows, _wfull, _full, _wfull, _full],
        out_specs=_rows,
        out_shape=jax.ShapeDtypeStruct((_N, _D), F32))(x, ws, bs, wd, bd)


def _cnt_recip(cnt):
    def body(cnt_ref, o_ref):
        s = cnt_ref[0, :_N, 0:1] + cnt_ref[1, :_N, 0:1]
        o_ref[...] = 1.0 / jnp.maximum(s, 1.0)

    return pl.pallas_call(
        body, out_shape=jax.ShapeDtypeStruct((_N, 1), F32))(cnt)


def _sage_lin(acc, recip, h, wl, bl, wr, with_stats):
    def body(acc_ref, rcp_ref, h_ref, wl_ref, bl_ref, wr_ref, *out_refs):
        o_ref = out_refs[0]
        a = acc_ref[0] + acc_ref[1]
        mean = a * rcp_ref[...]
        hp = (_dot_t(mean, wl_ref[...]) + bl_ref[...]
              + _dot_t(h_ref[...], wr_ref[...]))
        o_ref[...] = hp
        if with_stats:
            s1_ref, s2_ref = out_refs[1], out_refs[2]

            @pl.when(pl.program_id(0) == 0)
            def _():
                s1_ref[...] = jnp.zeros_like(s1_ref)
                s2_ref[...] = jnp.zeros_like(s2_ref)

            s1_ref[...] += jnp.sum(hp, axis=0, keepdims=True)
            s2_ref[...] += jnp.sum(hp * hp, axis=0, keepdims=True)

    out_shape = [jax.ShapeDtypeStruct((_N, _D), F32)]
    out_specs = [_rows]
    if with_stats:
        out_shape += [jax.ShapeDtypeStruct((1, _D), F32)] * 2
        out_specs += [_full, _full]
    return pl.pallas_call(
        body,
        grid=(_NB,),
        in_specs=[_acc_bs, _rcp_bs, _rows, _wfull, _full, _wfull],
        out_specs=out_specs,
        out_shape=out_shape)(acc, recip, h, wl, bl, wr)


def _bn_relu(hp, s1, s2, g, b):
    def body(hp_ref, s1_ref, s2_ref, g_ref, b_ref, o_ref):
        inv_n = 1.0 / _N
        mu = s1_ref[...] * inv_n
        var = s2_ref[...] * inv_n - mu * mu
        hn = (hp_ref[...] - mu) * lax.rsqrt(var + 1e-5) * g_ref[...] + b_ref[...]
        o_ref[...] = jnp.maximum(hn, 0.0)

    return pl.pallas_call(
        body,
        grid=(_NB,),
        in_specs=[_rows, _full, _full, _full, _full],
        out_specs=_rows,
        out_shape=jax.ShapeDtypeStruct((_N, _D), F32))(hp, s1, s2, g, b)


def _sc_mesh():
    return plsc.VectorSubcoreMesh(core_axis_name="c", subcore_axis_name="s")


_BB = 128
_SPANS = [(off, min(_BB, _RPT - off)) for off in range(0, _RPT, _BB)]


def _zero_acc(zrows_hbm, rows, acc_sh, row0):
    pltpu.sync_copy(zrows_hbm, rows)
    for off, sz in _SPANS:
        pltpu.sync_copy(rows.at[pl.ds(0, sz)],
                        acc_sh.at[pl.ds(row0 + off, sz)])


def _dump_acc(acc_sh, rows, out_hbm, row0, c):
    for off, sz in _SPANS:
        pltpu.sync_copy(acc_sh.at[pl.ds(row0 + off, sz)],
                        rows.at[pl.ds(0, sz)])
        pltpu.sync_copy(rows.at[pl.ds(0, sz)],
                        out_hbm.at[pl.ds(c * _NPAD + row0 + off, sz)])


def _make_seg_sum():
    @functools.partial(
        pl.kernel,
        mesh=_sc_mesh(),
        out_type=jax.ShapeDtypeStruct((2 * _NPAD, _D), F32),
        scratch_types=[
            pltpu.VMEM_SHARED((_NPAD, _D), F32),
            pltpu.VMEM((_CH,), jnp.int32),
            pltpu.VMEM((_CH,), jnp.int32),
            pltpu.VMEM((_CH, _D), F32),
            pltpu.SemaphoreType.DMA,
        ],
    )
    def seg(h_hbm, src_hbm, dst_hbm, zrows_hbm, acc_out,
            acc_sh, sidx, didx, rows, sem):
        c = lax.axis_index("c")
        s = lax.axis_index("s")
        row0 = s * _RPT
        fast = c == _FAST_C
        nch = jnp.where(fast, _NCF, _NCS)
        base = jnp.where(fast, s * (_NCF * _CH),
                         _NS * (_NCF * _CH) + s * (_NCS * _CH))

        _zero_acc(zrows_hbm, rows, acc_sh, row0)
        plsc.subcore_barrier()

        def chunk(i, carry):
            off = base + i * _CH
            pltpu.sync_copy(src_hbm.at[pl.ds(off, _CH)], sidx)
            pltpu.sync_copy(dst_hbm.at[pl.ds(off, _CH)], didx)
            pltpu.async_copy(h_hbm.at[sidx], rows, sem).wait()
            pltpu.sync_copy(rows, acc_sh.at[didx], add=True)
            return carry

        lax.fori_loop(0, nch, chunk, 0)
        plsc.subcore_barrier()
        _dump_acc(acc_sh, rows, acc_out, row0, c)

    return seg


def _make_cnt():
    @functools.partial(
        pl.kernel,
        mesh=_sc_mesh(),
        out_type=jax.ShapeDtypeStruct((2 * _NPAD, _D), F32),
        scratch_types=[
            pltpu.VMEM_SHARED((_NPAD, _D), F32),
            pltpu.VMEM((_CHC,), jnp.int32),
            pltpu.VMEM((_CHC, _D), F32),
            pltpu.VMEM((_BB, _D), F32),
        ],
    )
    def cntk(dst_hbm, zrows_hbm, ones_hbm, cnt_out,
             cnt_sh, didx, ones_v, rows):
        c = lax.axis_index("c")
        s = lax.axis_index("s")
        wid = s * _NC + c
        base = wid * _EPT
        row0 = s * _RPT

        _zero_acc(zrows_hbm, rows, cnt_sh, row0)
        pltpu.sync_copy(ones_hbm, ones_v)
        plsc.subcore_barrier()

        def chunkc(i, carry):
            pltpu.sync_copy(dst_hbm.at[pl.ds(base + i * _CHC, _CHC)], didx)
            pltpu.sync_copy(ones_v, cnt_sh.at[didx], add=True)
            return carry

        lax.fori_loop(0, _EPT // _CHC, chunkc, 0)
        plsc.subcore_barrier()
        _dump_acc(cnt_sh, rows, cnt_out, row0, c)

    return cntk


_seg_sum = _make_seg_sum()
_cnt_scatter = _make_cnt()


def kernel(x, edge_index, num_nodes_type_1, num_nodes_type_2,
           W_src, b_src, W_dst, b_dst,
           Wl0, bl0, Wr0, gamma0, beta0, Wl1, bl1, Wr1):
    del num_nodes_type_1, num_nodes_type_2
    src = edge_index[0]
    dst = edge_index[1]
    npad = _EPAD - _E
    src_p = jnp.concatenate([src, jnp.zeros((npad,), jnp.int32)])
    dst_p = jnp.concatenate([dst, jnp.full((npad,), _N, jnp.int32)])
    zrows = jnp.zeros((_BB, _D), F32)
    ones_r = jnp.ones((_CHC, _D), F32)

    bs = b_src.reshape(1, _D)
    bd = b_dst.reshape(1, _D)
    bl0r = bl0.reshape(1, _D)
    bl1r = bl1.reshape(1, _D)
    g0 = gamma0.reshape(1, _D)
    be0 = beta0.reshape(1, _D)

    h = _proj(x, W_src, bs, W_dst, bd)
    cnt = _cnt_scatter(dst_p, zrows, ones_r).reshape(2, _NPAD, _D)
    recip = _cnt_recip(cnt)
    acc0 = _seg_sum(h, src_p, dst_p, zrows).reshape(2, _NPAD, _D)
    hp0, s1, s2 = _sage_lin(acc0, recip, h, Wl0, bl0r, Wr0, with_stats=True)
    h2 = _bn_relu(hp0, s1, s2, g0, be0)
    acc1 = _seg_sum(h2, src_p, dst_p, zrows).reshape(2, _NPAD, _D)
    return _sage_lin(acc1, recip, h2, Wl1, bl1r, Wr1, with_stats=False)[0]


# speedup vs baseline: 1.0325x; 1.0189x over previous
import functools

import jax
import jax.numpy as jnp
from jax import lax
from jax.experimental import pallas as pl
from jax.experimental.pallas import tpu as pltpu
from jax.experimental.pallas import tpu_sc as plsc

F32 = jnp.float32

_N1 = 5000
_N2 = 5000
_N = _N1 + _N2
_E = 320000
_D = 128

_NC = 2
_NS = 16
_NW = _NC * _NS
_CH = 128
_CHC = 128
_NCF = 100
_NCS = 60
_FAST_C = 0
_EPAD = _NS * (_NCF + _NCS) * _CH
_EPT = _EPAD // _NW
_NPAD = _N + 112
_RPT = _NPAD // _NS


def _dot_t(a, w):
    return lax.dot_general(a, w, (((1,), (1,)), ((), ())),
                           precision=lax.Precision.HIGHEST,
                           preferred_element_type=F32)


_BR = 1000
_NB = _N // _BR

_full = pl.BlockSpec((1, _D), lambda i: (0, 0))
_wfull = pl.BlockSpec((_D, _D), lambda i: (0, 0))
_rows = pl.BlockSpec((_BR, _D), lambda i: (i, 0))
_acc_bs = pl.BlockSpec((2, _BR, _D), lambda i: (0, i, 0))
_rcp_bs = pl.BlockSpec((_BR, 1), lambda i: (i, 0))


def _proj(x, ws, bs, wd, bd):
    def body(x_ref, ws_ref, bs_ref, wd_ref, bd_ref, o_ref):
        i = pl.program_id(0)
        first = i < (_N1 // _BR)
        w = jnp.where(first, ws_ref[...], wd_ref[...])
        bias = jnp.where(first, bs_ref[...], bd_ref[...])
        o_ref[...] = _dot_t(x_ref[...], w) + bias

    return pl.pallas_call(
        body,
        grid=(_NB,),
        in_specs=[_rows, _wfull, _full, _wfull, _full],
        out_specs=_rows,
        out_shape=jax.ShapeDtypeStruct((_N, _D), F32))(x, ws, bs, wd, bd)


def _cnt_recip(cnt):
    def body(cnt_ref, o_ref):
        s = cnt_ref[0, :_N, 0:1] + cnt_ref[1, :_N, 0:1]
        o_ref[...] = 1.0 / jnp.maximum(s, 1.0)

    return pl.pallas_call(
        body, out_shape=jax.ShapeDtypeStruct((_N, 1), F32))(cnt)


def _sage_lin(acc, recip, h, wl, bl, wr, with_stats):
    def body(acc_ref, rcp_ref, h_ref, wl_ref, bl_ref, wr_ref, *out_refs):
        o_ref = out_refs[0]
        a = acc_ref[0] + acc_ref[1]
        mean = a * rcp_ref[...]
        hp = (_dot_t(mean, wl_ref[...]) + bl_ref[...]
              + _dot_t(h_ref[...], wr_ref[...]))
        o_ref[...] = hp
        if with_stats:
            s1_ref, s2_ref = out_refs[1], out_refs[2]

            @pl.when(pl.program_id(0) == 0)
            def _():
                s1_ref[...] = jnp.zeros_like(s1_ref)
                s2_ref[...] = jnp.zeros_like(s2_ref)

            s1_ref[...] += jnp.sum(hp, axis=0, keepdims=True)
            s2_ref[...] += jnp.sum(hp * hp, axis=0, keepdims=True)

    out_shape = [jax.ShapeDtypeStruct((_N, _D), F32)]
    out_specs = [_rows]
    if with_stats:
        out_shape += [jax.ShapeDtypeStruct((1, _D), F32)] * 2
        out_specs += [_full, _full]
    return pl.pallas_call(
        body,
        grid=(_NB,),
        in_specs=[_acc_bs, _rcp_bs, _rows, _wfull, _full, _wfull],
        out_specs=out_specs,
        out_shape=out_shape)(acc, recip, h, wl, bl, wr)


def _bn_relu(hp, s1, s2, g, b):
    def body(hp_ref, s1_ref, s2_ref, g_ref, b_ref, o_ref):
        inv_n = 1.0 / _N
        mu = s1_ref[...] * inv_n
        var = s2_ref[...] * inv_n - mu * mu
        hn = (hp_ref[...] - mu) * lax.rsqrt(var + 1e-5) * g_ref[...] + b_ref[...]
        o_ref[...] = jnp.maximum(hn, 0.0)

    return pl.pallas_call(
        body,
        grid=(_NB,),
        in_specs=[_rows, _full, _full, _full, _full],
        out_specs=_rows,
        out_shape=jax.ShapeDtypeStruct((_N, _D), F32))(hp, s1, s2, g, b)


def _sc_mesh():
    return plsc.VectorSubcoreMesh(core_axis_name="c", subcore_axis_name="s")


_BB = 128
_SPANS = [(off, min(_BB, _RPT - off)) for off in range(0, _RPT, _BB)]


def _zero_acc(zrows_hbm, rows, acc_sh, row0):
    pltpu.sync_copy(zrows_hbm, rows)
    for off, sz in _SPANS:
        pltpu.sync_copy(rows.at[pl.ds(0, sz)],
                        acc_sh.at[pl.ds(row0 + off, sz)])


def _dump_acc(acc_sh, rows, out_hbm, row0, c):
    for off, sz in _SPANS:
        pltpu.sync_copy(acc_sh.at[pl.ds(row0 + off, sz)],
                        rows.at[pl.ds(0, sz)])
        pltpu.sync_copy(rows.at[pl.ds(0, sz)],
                        out_hbm.at[pl.ds(c * _NPAD + row0 + off, sz)])


def _make_seg_sum():
    @functools.partial(
        pl.kernel,
        mesh=_sc_mesh(),
        out_type=jax.ShapeDtypeStruct((2 * _NPAD, _D), F32),
        scratch_types=[
            pltpu.VMEM_SHARED((_NPAD, _D), F32),
            pltpu.VMEM((_CH,), jnp.int32),
            pltpu.VMEM((_CH,), jnp.int32),
            pltpu.VMEM((_CH, _D), F32),
            pltpu.SemaphoreType.DMA,
        ],
    )
    def seg(h_hbm, src_hbm, dst_hbm, zrows_hbm, acc_out,
            acc_sh, sidx, didx, rows, sem):
        c = lax.axis_index("c")
        s = lax.axis_index("s")
        row0 = s * _RPT
        fast = c == _FAST_C
        nch = jnp.where(fast, _NCF, _NCS)
        base = jnp.where(fast, s * (_NCF * _CH),
                         _NS * (_NCF * _CH) + s * (_NCS * _CH))

        _zero_acc(zrows_hbm, rows, acc_sh, row0)
        plsc.subcore_barrier()

        def chunk(i, carry):
            off = base + i * _CH
            pltpu.sync_copy(src_hbm.at[pl.ds(off, _CH)], sidx)
            pltpu.sync_copy(dst_hbm.at[pl.ds(off, _CH)], didx)
            pltpu.async_copy(h_hbm.at[sidx], rows, sem).wait()
            pltpu.sync_copy(rows, acc_sh.at[didx], add=True)
            return carry

        lax.fori_loop(0, nch, chunk, 0)
        plsc.subcore_barrier()
        _dump_acc(acc_sh, rows, acc_out, row0, c)

    return seg


def _make_cnt():
    @functools.partial(
        pl.kernel,
        mesh=_sc_mesh(),
        out_type=jax.ShapeDtypeStruct((2 * _NPAD, _D), F32),
        scratch_types=[
            pltpu.VMEM_SHARED((_NPAD, _D), F32),
            pltpu.VMEM((_CHC,), jnp.int32),
            pltpu.VMEM((_CHC, _D), F32),
            pltpu.VMEM((_BB, _D), F32),
        ],
    )
    def cntk(dst_hbm, zrows_hbm, ones_hbm, cnt_out,
             cnt_sh, didx, ones_v, rows):
        c = lax.axis_index("c")
        s = lax.axis_index("s")
        wid = s * _NC + c
        base = wid * _EPT
        row0 = s * _RPT

        _zero_acc(zrows_hbm, rows, cnt_sh, row0)
        pltpu.sync_copy(ones_hbm, ones_v)
        plsc.subcore_barrier()

        def chunkc(i, carry):
            pltpu.sync_copy(dst_hbm.at[pl.ds(base + i * _CHC, _CHC)], didx)
            pltpu.sync_copy(ones_v, cnt_sh.at[didx], add=True)
            return carry

        lax.fori_loop(0, _EPT // _CHC, chunkc, 0)
        plsc.subcore_barrier()
        _dump_acc(cnt_sh, rows, cnt_out, row0, c)

    return cntk


_seg_sum = _make_seg_sum()
_cnt_scatter = _make_cnt()


def kernel(x, edge_index, num_nodes_type_1, num_nodes_type_2,
           W_src, b_src, W_dst, b_dst,
           Wl0, bl0, Wr0, gamma0, beta0, Wl1, bl1, Wr1):
    del num_nodes_type_1, num_nodes_type_2
    src = edge_index[0]
    dst = edge_index[1]
    npad = _EPAD - _E
    src_p = jnp.concatenate([src, jnp.zeros((npad,), jnp.int32)])
    dst_p = jnp.concatenate([dst, jnp.full((npad,), _N, jnp.int32)])
    zrows = jnp.zeros((_BB, _D), F32)
    ones_r = jnp.ones((_CHC, _D), F32)

    bs = b_src.reshape(1, _D)
    bd = b_dst.reshape(1, _D)
    bl0r = bl0.reshape(1, _D)
    bl1r = bl1.reshape(1, _D)
    g0 = gamma0.reshape(1, _D)
    be0 = beta0.reshape(1, _D)

    h = _proj(x, W_src, bs, W_dst, bd)
    cnt = _cnt_scatter(dst_p, zrows, ones_r).reshape(2, _NPAD, _D)
    recip = _cnt_recip(cnt)
    acc0 = _seg_sum(h, src_p, dst_p, zrows).reshape(2, _NPAD, _D)
    hp0, s1, s2 = _sage_lin(acc0, recip, h, Wl0, bl0r, Wr0, with_stats=True)
    h2 = _bn_relu(hp0, s1, s2, g0, be0)
    acc1 = _seg_sum(h2, src_p, dst_p, zrows).reshape(2, _NPAD, _D)
    return _sage_lin(acc1, recip, h2, Wl1, bl1r, Wr1, with_stats=False)[0]


# speedup vs baseline: 1.2521x; 1.2126x over previous
import functools

import jax
import jax.numpy as jnp
from jax import lax
from jax.experimental import pallas as pl
from jax.experimental.pallas import tpu as pltpu
from jax.experimental.pallas import tpu_sc as plsc

F32 = jnp.float32

_N1 = 5000
_N2 = 5000
_N = _N1 + _N2
_E = 320000
_D = 128

_NC = 2
_NS = 16
_NW = _NC * _NS
_CH = 128
_CHC = 128
_NCHUNK = 79
_EPT = _NCHUNK * _CH
_EPAD = _EPT * _NW
_NPAD = _N + 112
_RPT = _NPAD // _NS


def _dot_t(a, w):
    return lax.dot_general(a, w, (((1,), (1,)), ((), ())),
                           precision=lax.Precision.HIGHEST,
                           preferred_element_type=F32)


_BR = 1000
_NB = _N // _BR

_full = pl.BlockSpec((1, _D), lambda i: (0, 0))
_wfull = pl.BlockSpec((_D, _D), lambda i: (0, 0))
_rows = pl.BlockSpec((_BR, _D), lambda i: (i, 0))
_acc_bs = pl.BlockSpec((2, _BR, _D), lambda i: (0, i, 0))
_rcp_bs = pl.BlockSpec((_BR, 1), lambda i: (i, 0))


def _proj(x, ws, bs, wd, bd):
    def body(x_ref, ws_ref, bs_ref, wd_ref, bd_ref, o_ref):
        i = pl.program_id(0)
        first = i < (_N1 // _BR)
        w = jnp.where(first, ws_ref[...], wd_ref[...])
        bias = jnp.where(first, bs_ref[...], bd_ref[...])
        o_ref[...] = _dot_t(x_ref[...], w) + bias

    return pl.pallas_call(
        body,
        grid=(_NB,),
        in_specs=[_rows, _wfull, _full, _wfull, _full],
        out_specs=_rows,
        out_shape=jax.ShapeDtypeStruct((_N, _D), F32))(x, ws, bs, wd, bd)


def _cnt_recip(cnt):
    def body(cnt_ref, o_ref):
        s = cnt_ref[0, :_N, 0:1] + cnt_ref[1, :_N, 0:1]
        o_ref[...] = 1.0 / jnp.maximum(s, 1.0)

    return pl.pallas_call(
        body, out_shape=jax.ShapeDtypeStruct((_N, 1), F32))(cnt)


def _sage_lin(acc, recip, h, wl, bl, wr, with_stats):
    def body(acc_ref, rcp_ref, h_ref, wl_ref, bl_ref, wr_ref, *out_refs):
        o_ref = out_refs[0]
        a = acc_ref[0] + acc_ref[1]
        mean = a * rcp_ref[...]
        hp = (_dot_t(mean, wl_ref[...]) + bl_ref[...]
              + _dot_t(h_ref[...], wr_ref[...]))
        o_ref[...] = hp
        if with_stats:
            s1_ref, s2_ref = out_refs[1], out_refs[2]

            @pl.when(pl.program_id(0) == 0)
            def _():
                s1_ref[...] = jnp.zeros_like(s1_ref)
                s2_ref[...] = jnp.zeros_like(s2_ref)

            s1_ref[...] += jnp.sum(hp, axis=0, keepdims=True)
            s2_ref[...] += jnp.sum(hp * hp, axis=0, keepdims=True)

    out_shape = [jax.ShapeDtypeStruct((_N, _D), F32)]
    out_specs = [_rows]
    if with_stats:
        out_shape += [jax.ShapeDtypeStruct((1, _D), F32)] * 2
        out_specs += [_full, _full]
    return pl.pallas_call(
        body,
        grid=(_NB,),
        in_specs=[_acc_bs, _rcp_bs, _rows, _wfull, _full, _wfull],
        out_specs=out_specs,
        out_shape=out_shape)(acc, recip, h, wl, bl, wr)


def _bn_relu(hp, s1, s2, g, b):
    def body(hp_ref, s1_ref, s2_ref, g_ref, b_ref, o_ref):
        inv_n = 1.0 / _N
        mu = s1_ref[...] * inv_n
        var = s2_ref[...] * inv_n - mu * mu
        hn = (hp_ref[...] - mu) * lax.rsqrt(var + 1e-5) * g_ref[...] + b_ref[...]
        o_ref[...] = jnp.maximum(hn, 0.0)

    return pl.pallas_call(
        body,
        grid=(_NB,),
        in_specs=[_rows, _full, _full, _full, _full],
        out_specs=_rows,
        out_shape=jax.ShapeDtypeStruct((_N, _D), F32))(hp, s1, s2, g, b)


def _sc_mesh():
    return plsc.VectorSubcoreMesh(core_axis_name="c", subcore_axis_name="s")


_BB = 128
_SPANS = [(off, min(_BB, _RPT - off)) for off in range(0, _RPT, _BB)]


def _zero_acc(zrows_hbm, rows, acc_sh, row0):
    pltpu.sync_copy(zrows_hbm, rows)
    for off, sz in _SPANS:
        pltpu.sync_copy(rows.at[pl.ds(0, sz)],
                        acc_sh.at[pl.ds(row0 + off, sz)])


def _dump_acc(acc_sh, rows, out_hbm, row0, c):
    for off, sz in _SPANS:
        pltpu.sync_copy(acc_sh.at[pl.ds(row0 + off, sz)],
                        rows.at[pl.ds(0, sz)])
        pltpu.sync_copy(rows.at[pl.ds(0, sz)],
                        out_hbm.at[pl.ds(c * _NPAD + row0 + off, sz)])


def _make_seg_sum():
    @functools.partial(
        pl.kernel,
        mesh=_sc_mesh(),
        out_type=jax.ShapeDtypeStruct((2 * _NPAD, _D), F32),
        scratch_types=[
            pltpu.VMEM_SHARED((_NPAD, _D), F32),
            pltpu.VMEM((_CH,), jnp.int32),
            pltpu.VMEM((_CH,), jnp.int32),
            pltpu.VMEM((_CH, _D), F32),
            pltpu.SemaphoreType.DMA,
        ],
    )
    def seg(h_hbm, src_hbm, dst_hbm, zrows_hbm, acc_out,
            acc_sh, sidx, didx, rows, sem):
        c = lax.axis_index("c")
        s = lax.axis_index("s")
        wid = s * _NC + c
        base = wid * _EPT
        row0 = s * _RPT

        _zero_acc(zrows_hbm, rows, acc_sh, row0)
        plsc.subcore_barrier()

        def chunk(i, carry):
            off = base + i * _CH
            pltpu.sync_copy(src_hbm.at[pl.ds(off, _CH)], sidx)
            pltpu.sync_copy(dst_hbm.at[pl.ds(off, _CH)], didx)
            pltpu.async_copy(h_hbm.at[sidx], rows, sem).wait()
            pltpu.sync_copy(rows, acc_sh.at[didx], add=True)
            return carry

        lax.fori_loop(0, _NCHUNK, chunk, 0)
        plsc.subcore_barrier()
        _dump_acc(acc_sh, rows, acc_out, row0, c)

    return seg


def _make_cnt():
    @functools.partial(
        pl.kernel,
        mesh=_sc_mesh(),
        out_type=jax.ShapeDtypeStruct((2 * _NPAD, _D), F32),
        scratch_types=[
            pltpu.VMEM_SHARED((_NPAD, _D), F32),
            pltpu.VMEM((_CHC,), jnp.int32),
            pltpu.VMEM((_CHC, _D), F32),
            pltpu.VMEM((_BB, _D), F32),
        ],
    )
    def cntk(dst_hbm, zrows_hbm, ones_hbm, cnt_out,
             cnt_sh, didx, ones_v, rows):
        c = lax.axis_index("c")
        s = lax.axis_index("s")
        wid = s * _NC + c
        base = wid * _EPT
        row0 = s * _RPT

        _zero_acc(zrows_hbm, rows, cnt_sh, row0)
        pltpu.sync_copy(ones_hbm, ones_v)
        plsc.subcore_barrier()

        def chunkc(i, carry):
            pltpu.sync_copy(dst_hbm.at[pl.ds(base + i * _CHC, _CHC)], didx)
            pltpu.sync_copy(ones_v, cnt_sh.at[didx], add=True)
            return carry

        lax.fori_loop(0, _EPT // _CHC, chunkc, 0)
        plsc.subcore_barrier()
        _dump_acc(cnt_sh, rows, cnt_out, row0, c)

    return cntk


_seg_sum = _make_seg_sum()
_cnt_scatter = _make_cnt()


def kernel(x, edge_index, num_nodes_type_1, num_nodes_type_2,
           W_src, b_src, W_dst, b_dst,
           Wl0, bl0, Wr0, gamma0, beta0, Wl1, bl1, Wr1):
    del num_nodes_type_1, num_nodes_type_2
    src = edge_index[0]
    dst = edge_index[1]
    npad = _EPAD - _E
    src_p = jnp.concatenate([src, jnp.zeros((npad,), jnp.int32)])
    dst_p = jnp.concatenate([dst, jnp.full((npad,), _N, jnp.int32)])
    zrows = jnp.zeros((_BB, _D), F32)
    ones_r = jnp.ones((_CHC, _D), F32)

    bs = b_src.reshape(1, _D)
    bd = b_dst.reshape(1, _D)
    bl0r = bl0.reshape(1, _D)
    bl1r = bl1.reshape(1, _D)
    g0 = gamma0.reshape(1, _D)
    be0 = beta0.reshape(1, _D)

    h = _proj(x, W_src, bs, W_dst, bd)
    cnt = _cnt_scatter(dst_p, zrows, ones_r).reshape(2, _NPAD, _D)
    recip = _cnt_recip(cnt)
    acc0 = _seg_sum(h, src_p, dst_p, zrows).reshape(2, _NPAD, _D)
    hp0, s1, s2 = _sage_lin(acc0, recip, h, Wl0, bl0r, Wr0, with_stats=True)
    h2 = _bn_relu(hp0, s1, s2, g0, be0)
    acc1 = _seg_sum(h2, src_p, dst_p, zrows).reshape(2, _NPAD, _D)
    return _sage_lin(acc1, recip, h2, Wl1, bl1r, Wr1, with_stats=False)[0]


# speedup vs baseline: 1.3886x; 1.1090x over previous
import functools

import jax
import jax.numpy as jnp
from jax import lax
from jax.experimental import pallas as pl
from jax.experimental.pallas import tpu as pltpu
from jax.experimental.pallas import tpu_sc as plsc

F32 = jnp.float32

_N1 = 5000
_N2 = 5000
_N = _N1 + _N2
_E = 320000
_D = 128

_NC = 2
_NS = 16
_NW = _NC * _NS
_CH = 128
_CHC = 128
_NCHUNK = 79
_EPT = _NCHUNK * _CH
_EPAD = _EPT * _NW
_NPAD = _N + 240
_RPT = _NPAD // _NS


def _dot_t(a, w):
    return lax.dot_general(a, w, (((1,), (1,)), ((), ())),
                           precision=lax.Precision.HIGHEST,
                           preferred_element_type=F32)


_BR = 1000
_NB = _N // _BR

_full = pl.BlockSpec((1, _D), lambda i: (0, 0))
_wfull = pl.BlockSpec((_D, _D), lambda i: (0, 0))
_rows = pl.BlockSpec((_BR, _D), lambda i: (i, 0))
_acc_bs = pl.BlockSpec((2, _BR, _D), lambda i: (0, i, 0))
_rcp_bs = pl.BlockSpec((_BR, 1), lambda i: (i, 0))


def _proj(x, ws, bs, wd, bd):
    def body(x_ref, ws_ref, bs_ref, wd_ref, bd_ref, o_ref):
        i = pl.program_id(0)
        first = i < (_N1 // _BR)
        w = jnp.where(first, ws_ref[...], wd_ref[...])
        bias = jnp.where(first, bs_ref[...], bd_ref[...])
        o_ref[...] = _dot_t(x_ref[...], w) + bias

    return pl.pallas_call(
        body,
        grid=(_NB,),
        in_specs=[_rows, _wfull, _full, _wfull, _full],
        out_specs=_rows,
        out_shape=jax.ShapeDtypeStruct((_N, _D), F32))(x, ws, bs, wd, bd)


def _cnt_recip(cnt):
    def body(cnt_ref, o_ref):
        s = (cnt_ref[0, :_N, 0:1] + cnt_ref[1, :_N, 0:1]).astype(F32)
        o_ref[...] = 1.0 / jnp.maximum(s, 1.0)

    return pl.pallas_call(
        body, out_shape=jax.ShapeDtypeStruct((_N, 1), F32))(cnt)


def _sage_lin(acc, recip, h, wl, bl, wr, with_stats):
    def body(acc_ref, rcp_ref, h_ref, wl_ref, bl_ref, wr_ref, *out_refs):
        o_ref = out_refs[0]
        a = acc_ref[0] + acc_ref[1]
        mean = a * rcp_ref[...]
        hp = (_dot_t(mean, wl_ref[...]) + bl_ref[...]
              + _dot_t(h_ref[...], wr_ref[...]))
        o_ref[...] = hp
        if with_stats:
            s1_ref, s2_ref = out_refs[1], out_refs[2]

            @pl.when(pl.program_id(0) == 0)
            def _():
                s1_ref[...] = jnp.zeros_like(s1_ref)
                s2_ref[...] = jnp.zeros_like(s2_ref)

            s1_ref[...] += jnp.sum(hp, axis=0, keepdims=True)
            s2_ref[...] += jnp.sum(hp * hp, axis=0, keepdims=True)

    out_shape = [jax.ShapeDtypeStruct((_N, _D), F32)]
    out_specs = [_rows]
    if with_stats:
        out_shape += [jax.ShapeDtypeStruct((1, _D), F32)] * 2
        out_specs += [_full, _full]
    return pl.pallas_call(
        body,
        grid=(_NB,),
        in_specs=[_acc_bs, _rcp_bs, _rows, _wfull, _full, _wfull],
        out_specs=out_specs,
        out_shape=out_shape)(acc, recip, h, wl, bl, wr)


def _bn_relu(hp, s1, s2, g, b):
    def body(hp_ref, s1_ref, s2_ref, g_ref, b_ref, o_ref):
        inv_n = 1.0 / _N
        mu = s1_ref[...] * inv_n
        var = s2_ref[...] * inv_n - mu * mu
        hn = (hp_ref[...] - mu) * lax.rsqrt(var + 1e-5) * g_ref[...] + b_ref[...]
        o_ref[...] = jnp.maximum(hn, 0.0)

    return pl.pallas_call(
        body,
        grid=(_NB,),
        in_specs=[_rows, _full, _full, _full, _full],
        out_specs=_rows,
        out_shape=jax.ShapeDtypeStruct((_N, _D), F32))(hp, s1, s2, g, b)


def _sc_mesh():
    return plsc.VectorSubcoreMesh(core_axis_name="c", subcore_axis_name="s")


_BB = 128
_SPANS = [(off, min(_BB, _RPT - off)) for off in range(0, _RPT, _BB)]


def _zero_acc(zrows_hbm, rows, acc_sh, row0):
    pltpu.sync_copy(zrows_hbm, rows)
    for off, sz in _SPANS:
        pltpu.sync_copy(rows.at[pl.ds(0, sz)],
                        acc_sh.at[pl.ds(row0 + off, sz)])


def _dump_acc(acc_sh, rows, out_hbm, row0, c):
    for off, sz in _SPANS:
        pltpu.sync_copy(acc_sh.at[pl.ds(row0 + off, sz)],
                        rows.at[pl.ds(0, sz)])
        pltpu.sync_copy(rows.at[pl.ds(0, sz)],
                        out_hbm.at[pl.ds(c * _NPAD + row0 + off, sz)])


def _make_seg_sum():
    @functools.partial(
        pl.kernel,
        mesh=_sc_mesh(),
        out_type=jax.ShapeDtypeStruct((2 * _NPAD, _D), F32),
        scratch_types=[
            pltpu.VMEM_SHARED((_NPAD, _D), F32),
            pltpu.VMEM((_CH,), jnp.int32),
            pltpu.VMEM((_CH,), jnp.int32),
            pltpu.VMEM((_CH,), jnp.int32),
            pltpu.VMEM((_CH,), jnp.int32),
            pltpu.VMEM((_CH, _D), F32),
            pltpu.VMEM((_CH, _D), F32),
            pltpu.SemaphoreType.DMA,
            pltpu.SemaphoreType.DMA,
            pltpu.SemaphoreType.DMA,
        ],
    )
    def seg(h_hbm, src_hbm, dst_hbm, zrows_hbm, pad_hbm, acc_out,
            acc_sh, sidxA, didxA, sidxB, didxB, rowsA, rowsB,
            semG, semSA, semSB):
        c = lax.axis_index("c")
        s = lax.axis_index("s")
        wid = s * _NC + c
        base = wid * _EPT
        row0 = s * _RPT

        _zero_acc(zrows_hbm, rowsA, acc_sh, row0)
        pltpu.sync_copy(pad_hbm, didxA)
        pltpu.sync_copy(pad_hbm, didxB)
        plsc.subcore_barrier()
        pltpu.async_copy(rowsA, acc_sh.at[didxA], semSA, add=True)
        pltpu.async_copy(rowsA, acc_sh.at[didxB], semSB, add=True)

        def _step(ci, sidx, didx, rows, semS):
            pltpu.make_async_copy(rows, acc_sh.at[didx], semS).wait()
            off = base + ci * _CH
            pltpu.sync_copy(src_hbm.at[pl.ds(off, _CH)], sidx)
            pltpu.sync_copy(dst_hbm.at[pl.ds(off, _CH)], didx)
            pltpu.async_copy(h_hbm.at[sidx], rows, semG).wait()
            pltpu.async_copy(rows, acc_sh.at[didx], semS, add=True)

        def pair(j, carry):
            _step(2 * j, sidxA, didxA, rowsA, semSA)
            _step(2 * j + 1, sidxB, didxB, rowsB, semSB)
            return carry

        lax.fori_loop(0, (_NCHUNK - 1) // 2, pair, 0)
        _step(_NCHUNK - 1, sidxA, didxA, rowsA, semSA)
        pltpu.make_async_copy(rowsA, acc_sh.at[didxA], semSA).wait()
        pltpu.make_async_copy(rowsB, acc_sh.at[didxB], semSB).wait()
        plsc.subcore_barrier()
        _dump_acc(acc_sh, rowsA, acc_out, row0, c)

    return seg


def _make_cnt():
    @functools.partial(
        pl.kernel,
        mesh=_sc_mesh(),
        out_type=jax.ShapeDtypeStruct((2 * _NPAD, _D), F32),
        scratch_types=[
            pltpu.VMEM_SHARED((_NPAD, _D), F32),
            pltpu.VMEM((_CHC,), jnp.int32),
            pltpu.VMEM((_CHC, _D), F32),
            pltpu.VMEM((_BB, _D), F32),
        ],
    )
    def cntk(dst_hbm, zrows_hbm, ones_hbm, cnt_out,
             cnt_sh, didx, ones_v, rows):
        c = lax.axis_index("c")
        s = lax.axis_index("s")
        wid = s * _NC + c
        base = wid * _EPT
        row0 = s * _RPT

        _zero_acc(zrows_hbm, rows, cnt_sh, row0)
        pltpu.sync_copy(ones_hbm, ones_v)
        plsc.subcore_barrier()

        def chunkc(i, carry):
            pltpu.sync_copy(dst_hbm.at[pl.ds(base + i * _CHC, _CHC)], didx)
            pltpu.sync_copy(ones_v, cnt_sh.at[didx], add=True)
            return carry

        lax.fori_loop(0, _EPT // _CHC, chunkc, 0)
        plsc.subcore_barrier()
        _dump_acc(cnt_sh, rows, cnt_out, row0, c)

    return cntk


_seg_sum = _make_seg_sum()
_cnt_scatter = _make_cnt()


def kernel(x, edge_index, num_nodes_type_1, num_nodes_type_2,
           W_src, b_src, W_dst, b_dst,
           Wl0, bl0, Wr0, gamma0, beta0, Wl1, bl1, Wr1):
    del num_nodes_type_1, num_nodes_type_2
    src = edge_index[0]
    dst = edge_index[1]
    npad = _EPAD - _E
    src_p = jnp.concatenate([src, jnp.zeros((npad,), jnp.int32)])
    dst_p = jnp.concatenate([dst, jnp.full((npad,), _N, jnp.int32)])
    zrows = jnp.zeros((_BB, _D), F32)
    ones_r = jnp.ones((_CHC, _D), F32)
    pad_i = jnp.full((_CH,), _N, jnp.int32)

    bs = b_src.reshape(1, _D)
    bd = b_dst.reshape(1, _D)
    bl0r = bl0.reshape(1, _D)
    bl1r = bl1.reshape(1, _D)
    g0 = gamma0.reshape(1, _D)
    be0 = beta0.reshape(1, _D)

    h = _proj(x, W_src, bs, W_dst, bd)
    cnt = _cnt_scatter(dst_p, zrows, ones_r).reshape(2, _NPAD, _D)
    recip = _cnt_recip(cnt)
    acc0 = _seg_sum(h, src_p, dst_p, zrows, pad_i).reshape(2, _NPAD, _D)
    hp0, s1, s2 = _sage_lin(acc0, recip, h, Wl0, bl0r, Wr0, with_stats=True)
    h2 = _bn_relu(hp0, s1, s2, g0, be0)
    acc1 = _seg_sum(h2, src_p, dst_p, zrows, pad_i).reshape(2, _NPAD, _D)
    return _sage_lin(acc1, recip, h2, Wl1, bl1r, Wr1, with_stats=False)[0]


# speedup vs baseline: 1.4936x; 1.0756x over previous
import functools

import jax
import jax.numpy as jnp
from jax import lax
from jax.experimental import pallas as pl
from jax.experimental.pallas import tpu as pltpu
from jax.experimental.pallas import tpu_sc as plsc

F32 = jnp.float32

_N1 = 5000
_N2 = 5000
_N = _N1 + _N2
_E = 320000
_D = 128

_NC = 2
_NS = 16
_NW = _NC * _NS
_CH = 128
_CHC = 128
_NCHUNK = 79
_EPT = _NCHUNK * _CH
_EPAD = _EPT * _NW
_NPAD = _N + 240
_RPT = _NPAD // _NS


def _dot_t(a, w):
    return lax.dot_general(a, w, (((1,), (1,)), ((), ())),
                           precision=lax.Precision.HIGHEST,
                           preferred_element_type=F32)


_BR = 1000
_NB = _N // _BR

_full = pl.BlockSpec((1, _D), lambda i: (0, 0))
_wfull = pl.BlockSpec((_D, _D), lambda i: (0, 0))
_rows = pl.BlockSpec((_BR, _D), lambda i: (i, 0))
_acc_bs = pl.BlockSpec((2, _BR, _D), lambda i: (0, i, 0))
_rcp_bs = pl.BlockSpec((_BR, 1), lambda i: (i, 0))


def _proj(x, ws, bs, wd, bd):
    def body(x_ref, ws_ref, bs_ref, wd_ref, bd_ref, o_ref):
        i = pl.program_id(0)
        first = i < (_N1 // _BR)
        w = jnp.where(first, ws_ref[...], wd_ref[...])
        bias = jnp.where(first, bs_ref[...], bd_ref[...])
        o_ref[...] = _dot_t(x_ref[...], w) + bias

    return pl.pallas_call(
        body,
        grid=(_NB,),
        in_specs=[_rows, _wfull, _full, _wfull, _full],
        out_specs=_rows,
        out_shape=jax.ShapeDtypeStruct((_N, _D), F32))(x, ws, bs, wd, bd)


def _cnt_recip(cnt):
    def body(cnt_ref, o_ref):
        s = (cnt_ref[0, :_N, 0:1] + cnt_ref[1, :_N, 0:1]).astype(F32)
        o_ref[...] = 1.0 / jnp.maximum(s, 1.0)

    return pl.pallas_call(
        body, out_shape=jax.ShapeDtypeStruct((_N, 1), F32))(cnt)


def _sage_lin(acc, recip, h, wl, bl, wr, with_stats):
    def body(acc_ref, rcp_ref, h_ref, wl_ref, bl_ref, wr_ref, *out_refs):
        o_ref = out_refs[0]
        a = acc_ref[0] + acc_ref[1]
        mean = a * rcp_ref[...]
        hp = (_dot_t(mean, wl_ref[...]) + bl_ref[...]
              + _dot_t(h_ref[...], wr_ref[...]))
        o_ref[...] = hp
        if with_stats:
            s1_ref, s2_ref = out_refs[1], out_refs[2]

            @pl.when(pl.program_id(0) == 0)
            def _():
                s1_ref[...] = jnp.zeros_like(s1_ref)
                s2_ref[...] = jnp.zeros_like(s2_ref)

            s1_ref[...] += jnp.sum(hp, axis=0, keepdims=True)
            s2_ref[...] += jnp.sum(hp * hp, axis=0, keepdims=True)

    out_shape = [jax.ShapeDtypeStruct((_N, _D), F32)]
    out_specs = [_rows]
    if with_stats:
        out_shape += [jax.ShapeDtypeStruct((1, _D), F32)] * 2
        out_specs += [_full, _full]
    return pl.pallas_call(
        body,
        grid=(_NB,),
        in_specs=[_acc_bs, _rcp_bs, _rows, _wfull, _full, _wfull],
        out_specs=out_specs,
        out_shape=out_shape)(acc, recip, h, wl, bl, wr)


def _bn_relu(hp, s1, s2, g, b):
    def body(hp_ref, s1_ref, s2_ref, g_ref, b_ref, o_ref):
        inv_n = 1.0 / _N
        mu = s1_ref[...] * inv_n
        var = s2_ref[...] * inv_n - mu * mu
        hn = (hp_ref[...] - mu) * lax.rsqrt(var + 1e-5) * g_ref[...] + b_ref[...]
        o_ref[...] = jnp.maximum(hn, 0.0)

    return pl.pallas_call(
        body,
        grid=(_NB,),
        in_specs=[_rows, _full, _full, _full, _full],
        out_specs=_rows,
        out_shape=jax.ShapeDtypeStruct((_N, _D), F32))(hp, s1, s2, g, b)


def _sc_mesh():
    return plsc.VectorSubcoreMesh(core_axis_name="c", subcore_axis_name="s")


_BB = 128
_SPANS = [(off, min(_BB, _RPT - off)) for off in range(0, _RPT, _BB)]


def _zero_acc(zrows_hbm, rows, acc_sh, row0):
    pltpu.sync_copy(zrows_hbm, rows)
    for off, sz in _SPANS:
        pltpu.sync_copy(rows.at[pl.ds(0, sz)],
                        acc_sh.at[pl.ds(row0 + off, sz)])


def _dump_acc(acc_sh, rows, out_hbm, row0, c):
    for off, sz in _SPANS:
        pltpu.sync_copy(acc_sh.at[pl.ds(row0 + off, sz)],
                        rows.at[pl.ds(0, sz)])
        pltpu.sync_copy(rows.at[pl.ds(0, sz)],
                        out_hbm.at[pl.ds(c * _NPAD + row0 + off, sz)])


def _make_seg_sum():
    @functools.partial(
        pl.kernel,
        mesh=_sc_mesh(),
        out_type=jax.ShapeDtypeStruct((2 * _NPAD, _D), F32),
        scratch_types=[
            pltpu.VMEM_SHARED((_NPAD, _D), F32),
            pltpu.VMEM((_CH,), jnp.int32),
            pltpu.VMEM((_CH,), jnp.int32),
            pltpu.VMEM((_CH,), jnp.int32),
            pltpu.VMEM((_CH,), jnp.int32),
            pltpu.VMEM((_CH, _D), F32),
            pltpu.VMEM((_CH, _D), F32),
            pltpu.SemaphoreType.DMA,
            pltpu.SemaphoreType.DMA,
            pltpu.SemaphoreType.DMA,
            pltpu.SemaphoreType.DMA,
        ],
    )
    def seg(h_hbm, src_hbm, dst_hbm, zrows_hbm, pad_hbm, acc_out,
            acc_sh, sidxA, didxA, sidxB, didxB, rowsA, rowsB,
            semG, semSA, semSB, semI):
        c = lax.axis_index("c")
        s = lax.axis_index("s")
        wid = s * _NC + c
        base = wid * _EPT
        row0 = s * _RPT

        _zero_acc(zrows_hbm, rowsA, acc_sh, row0)
        pltpu.sync_copy(pad_hbm, didxA)
        pltpu.sync_copy(pad_hbm, didxB)
        plsc.subcore_barrier()
        pltpu.async_copy(rowsA, acc_sh.at[didxA], semSA, add=True)
        pltpu.async_copy(rowsA, acc_sh.at[didxB], semSB, add=True)

        def _step(ci, sidx, didx, rows, semS):
            off = base + ci * _CH
            cps = pltpu.async_copy(src_hbm.at[pl.ds(off, _CH)], sidx, semI)
            pltpu.make_async_copy(rows, acc_sh.at[didx], semS).wait()
            cpd = pltpu.async_copy(dst_hbm.at[pl.ds(off, _CH)], didx, semI)
            cps.wait()
            cpd.wait()
            pltpu.async_copy(h_hbm.at[sidx], rows, semG).wait()
            pltpu.async_copy(rows, acc_sh.at[didx], semS, add=True)

        def pair(j, carry):
            _step(2 * j, sidxA, didxA, rowsA, semSA)
            _step(2 * j + 1, sidxB, didxB, rowsB, semSB)
            return carry

        lax.fori_loop(0, (_NCHUNK - 1) // 2, pair, 0)
        _step(_NCHUNK - 1, sidxA, didxA, rowsA, semSA)
        pltpu.make_async_copy(rowsA, acc_sh.at[didxA], semSA).wait()
        pltpu.make_async_copy(rowsB, acc_sh.at[didxB], semSB).wait()
        plsc.subcore_barrier()
        _dump_acc(acc_sh, rowsA, acc_out, row0, c)

    return seg


def _make_cnt():
    @functools.partial(
        pl.kernel,
        mesh=_sc_mesh(),
        out_type=jax.ShapeDtypeStruct((2 * _NPAD, _D), F32),
        scratch_types=[
            pltpu.VMEM_SHARED((_NPAD, _D), F32),
            pltpu.VMEM((_CHC,), jnp.int32),
            pltpu.VMEM((_CHC, _D), F32),
            pltpu.VMEM((_BB, _D), F32),
        ],
    )
    def cntk(dst_hbm, zrows_hbm, ones_hbm, cnt_out,
             cnt_sh, didx, ones_v, rows):
        c = lax.axis_index("c")
        s = lax.axis_index("s")
        wid = s * _NC + c
        base = wid * _EPT
        row0 = s * _RPT

        _zero_acc(zrows_hbm, rows, cnt_sh, row0)
        pltpu.sync_copy(ones_hbm, ones_v)
        plsc.subcore_barrier()

        def chunkc(i, carry):
            pltpu.sync_copy(dst_hbm.at[pl.ds(base + i * _CHC, _CHC)], didx)
            pltpu.sync_copy(ones_v, cnt_sh.at[didx], add=True)
            return carry

        lax.fori_loop(0, _EPT // _CHC, chunkc, 0)
        plsc.subcore_barrier()
        _dump_acc(cnt_sh, rows, cnt_out, row0, c)

    return cntk


_seg_sum = _make_seg_sum()
_cnt_scatter = _make_cnt()


def kernel(x, edge_index, num_nodes_type_1, num_nodes_type_2,
           W_src, b_src, W_dst, b_dst,
           Wl0, bl0, Wr0, gamma0, beta0, Wl1, bl1, Wr1):
    del num_nodes_type_1, num_nodes_type_2
    src = edge_index[0]
    dst = edge_index[1]
    npad = _EPAD - _E
    src_p = jnp.concatenate([src, jnp.zeros((npad,), jnp.int32)])
    dst_p = jnp.concatenate([dst, jnp.full((npad,), _N, jnp.int32)])
    zrows = jnp.zeros((_BB, _D), F32)
    ones_r = jnp.ones((_CHC, _D), F32)
    pad_i = jnp.full((_CH,), _N, jnp.int32)

    bs = b_src.reshape(1, _D)
    bd = b_dst.reshape(1, _D)
    bl0r = bl0.reshape(1, _D)
    bl1r = bl1.reshape(1, _D)
    g0 = gamma0.reshape(1, _D)
    be0 = beta0.reshape(1, _D)

    h = _proj(x, W_src, bs, W_dst, bd)
    cnt = _cnt_scatter(dst_p, zrows, ones_r).reshape(2, _NPAD, _D)
    recip = _cnt_recip(cnt)
    acc0 = _seg_sum(h, src_p, dst_p, zrows, pad_i).reshape(2, _NPAD, _D)
    hp0, s1, s2 = _sage_lin(acc0, recip, h, Wl0, bl0r, Wr0, with_stats=True)
    h2 = _bn_relu(hp0, s1, s2, g0, be0)
    acc1 = _seg_sum(h2, src_p, dst_p, zrows, pad_i).reshape(2, _NPAD, _D)
    return _sage_lin(acc1, recip, h2, Wl1, bl1r, Wr1, with_stats=False)[0]


# speedup vs baseline: 1.5197x; 1.0175x over previous
import functools

import jax
import jax.numpy as jnp
from jax import lax
from jax.experimental import pallas as pl
from jax.experimental.pallas import tpu as pltpu
from jax.experimental.pallas import tpu_sc as plsc

F32 = jnp.float32

_N1 = 5000
_N2 = 5000
_N = _N1 + _N2
_E = 320000
_D = 128

_NC = 2
_NS = 16
_NW = _NC * _NS
_CH = 128
_CHC = 128
_NCHUNK = 79
_EPT = _NCHUNK * _CH
_EPAD = _EPT * _NW
_NPAD = _N + 240
_RPT = _NPAD // _NS


def _dot_t(a, w):
    return lax.dot_general(a, w, (((1,), (1,)), ((), ())),
                           precision=lax.Precision.HIGHEST,
                           preferred_element_type=F32)


_BR = 1000
_NB = _N // _BR

_full = pl.BlockSpec((1, _D), lambda i: (0, 0))
_wfull = pl.BlockSpec((_D, _D), lambda i: (0, 0))
_rows = pl.BlockSpec((_BR, _D), lambda i: (i, 0))
_acc_bs = pl.BlockSpec((2, _BR, _D), lambda i: (0, i, 0))
_rcp_bs = pl.BlockSpec((_BR, 1), lambda i: (i, 0))


def _proj(x, ws, bs, wd, bd):
    def body(x_ref, ws_ref, bs_ref, wd_ref, bd_ref, o_ref):
        i = pl.program_id(0)
        first = i < (_N1 // _BR)
        w = jnp.where(first, ws_ref[...], wd_ref[...])
        bias = jnp.where(first, bs_ref[...], bd_ref[...])
        o_ref[...] = _dot_t(x_ref[...], w) + bias

    return pl.pallas_call(
        body,
        grid=(_NB,),
        in_specs=[_rows, _wfull, _full, _wfull, _full],
        out_specs=_rows,
        out_shape=jax.ShapeDtypeStruct((_N, _D), F32))(x, ws, bs, wd, bd)


def _cnt_recip(cnt):
    def body(cnt_ref, o_ref):
        s = (cnt_ref[0, :_N, 0:1] + cnt_ref[1, :_N, 0:1]).astype(F32)
        o_ref[...] = 1.0 / jnp.maximum(s, 1.0)

    return pl.pallas_call(
        body, out_shape=jax.ShapeDtypeStruct((_N, 1), F32))(cnt)


def _sage_lin(acc, recip, h, wl, bl, wr, with_stats):
    def body(acc_ref, rcp_ref, h_ref, wl_ref, bl_ref, wr_ref, *out_refs):
        o_ref = out_refs[0]
        a = acc_ref[0] + acc_ref[1]
        mean = a * rcp_ref[...]
        hp = (_dot_t(mean, wl_ref[...]) + bl_ref[...]
              + _dot_t(h_ref[...], wr_ref[...]))
        o_ref[...] = hp
        if with_stats:
            s1_ref, s2_ref = out_refs[1], out_refs[2]

            @pl.when(pl.program_id(0) == 0)
            def _():
                s1_ref[...] = jnp.zeros_like(s1_ref)
                s2_ref[...] = jnp.zeros_like(s2_ref)

            s1_ref[...] += jnp.sum(hp, axis=0, keepdims=True)
            s2_ref[...] += jnp.sum(hp * hp, axis=0, keepdims=True)

    out_shape = [jax.ShapeDtypeStruct((_N, _D), F32)]
    out_specs = [_rows]
    if with_stats:
        out_shape += [jax.ShapeDtypeStruct((1, _D), F32)] * 2
        out_specs += [_full, _full]
    return pl.pallas_call(
        body,
        grid=(_NB,),
        in_specs=[_acc_bs, _rcp_bs, _rows, _wfull, _full, _wfull],
        out_specs=out_specs,
        out_shape=out_shape)(acc, recip, h, wl, bl, wr)


def _bn_relu(hp, s1, s2, g, b):
    def body(hp_ref, s1_ref, s2_ref, g_ref, b_ref, o_ref):
        inv_n = 1.0 / _N
        mu = s1_ref[...] * inv_n
        var = s2_ref[...] * inv_n - mu * mu
        hn = (hp_ref[...] - mu) * lax.rsqrt(var + 1e-5) * g_ref[...] + b_ref[...]
        o_ref[...] = jnp.maximum(hn, 0.0)

    return pl.pallas_call(
        body,
        grid=(_NB,),
        in_specs=[_rows, _full, _full, _full, _full],
        out_specs=_rows,
        out_shape=jax.ShapeDtypeStruct((_N, _D), F32))(hp, s1, s2, g, b)


def _sc_mesh():
    return plsc.VectorSubcoreMesh(core_axis_name="c", subcore_axis_name="s")


_BB = 128
_SPANS = [(off, min(_BB, _RPT - off)) for off in range(0, _RPT, _BB)]


def _zero_acc(zrows_hbm, rows, acc_sh, row0):
    pltpu.sync_copy(zrows_hbm, rows)
    for off, sz in _SPANS:
        pltpu.sync_copy(rows.at[pl.ds(0, sz)],
                        acc_sh.at[pl.ds(row0 + off, sz)])


def _dump_acc(acc_sh, rows, out_hbm, row0, c):
    for off, sz in _SPANS:
        pltpu.sync_copy(acc_sh.at[pl.ds(row0 + off, sz)],
                        rows.at[pl.ds(0, sz)])
        pltpu.sync_copy(rows.at[pl.ds(0, sz)],
                        out_hbm.at[pl.ds(c * _NPAD + row0 + off, sz)])


def _make_seg_sum():
    @functools.partial(
        pl.kernel,
        mesh=_sc_mesh(),
        out_type=jax.ShapeDtypeStruct((2 * _NPAD, _D), F32),
        scratch_types=[
            pltpu.VMEM_SHARED((_NPAD, _D), F32),
            pltpu.VMEM((_CH,), jnp.int32),
            pltpu.VMEM((_CH,), jnp.int32),
            pltpu.VMEM((_CH,), jnp.int32),
            pltpu.VMEM((_CH,), jnp.int32),
            pltpu.VMEM((_CH, _D), F32),
            pltpu.VMEM((_CH, _D), F32),
            pltpu.SemaphoreType.DMA,
            pltpu.SemaphoreType.DMA,
            pltpu.SemaphoreType.DMA,
            pltpu.SemaphoreType.DMA,
        ],
    )
    def seg(h_hbm, src_hbm, dst_hbm, zrows_hbm, pad_hbm, acc_out,
            acc_sh, sidxA, didxA, sidxB, didxB, rowsA, rowsB,
            semG, semSA, semSB, semI):
        c = lax.axis_index("c")
        s = lax.axis_index("s")
        wid = s * _NC + c
        base = wid * _EPT
        row0 = s * _RPT

        _zero_acc(zrows_hbm, rowsA, acc_sh, row0)
        pltpu.sync_copy(src_hbm.at[pl.ds(base, _CH)], sidxA)
        pltpu.sync_copy(dst_hbm.at[pl.ds(base, _CH)], didxA)
        pltpu.sync_copy(pad_hbm, didxB)
        plsc.subcore_barrier()
        pltpu.async_copy(rowsB, acc_sh.at[didxB], semSB, add=True)
        pltpu.async_copy(h_hbm.at[sidxA], rowsA, semG)

        def _step2(ci, sx_c, dx_c, rw_c, s_c, sx_o, dx_o, rw_o, s_o):
            pltpu.make_async_copy(h_hbm.at[sx_c], rw_c, semG).wait()
            pltpu.async_copy(rw_c, acc_sh.at[dx_c], s_c, add=True)
            off = base + (ci + 1) * _CH
            cps = pltpu.async_copy(src_hbm.at[pl.ds(off, _CH)], sx_o, semI)
            pltpu.make_async_copy(rw_o, acc_sh.at[dx_o], s_o).wait()
            cpd = pltpu.async_copy(dst_hbm.at[pl.ds(off, _CH)], dx_o, semI)
            cps.wait()
            cpd.wait()
            pltpu.async_copy(h_hbm.at[sx_o], rw_o, semG)

        def pair(j, carry):
            _step2(2 * j, sidxA, didxA, rowsA, semSA,
                   sidxB, didxB, rowsB, semSB)
            _step2(2 * j + 1, sidxB, didxB, rowsB, semSB,
                   sidxA, didxA, rowsA, semSA)
            return carry

        lax.fori_loop(0, (_NCHUNK - 1) // 2, pair, 0)
        _step2(_NCHUNK - 1, sidxA, didxA, rowsA, semSA,
               sidxB, didxB, rowsB, semSB)
        pltpu.make_async_copy(h_hbm.at[sidxB], rowsB, semG).wait()
        pltpu.make_async_copy(rowsA, acc_sh.at[didxA], semSA).wait()
        plsc.subcore_barrier()
        _dump_acc(acc_sh, rowsA, acc_out, row0, c)

    return seg


def _make_cnt():
    @functools.partial(
        pl.kernel,
        mesh=_sc_mesh(),
        out_type=jax.ShapeDtypeStruct((2 * _NPAD, _D), F32),
        scratch_types=[
            pltpu.VMEM_SHARED((_NPAD, _D), F32),
            pltpu.VMEM((_CHC,), jnp.int32),
            pltpu.VMEM((_CHC, _D), F32),
            pltpu.VMEM((_BB, _D), F32),
        ],
    )
    def cntk(dst_hbm, zrows_hbm, ones_hbm, cnt_out,
             cnt_sh, didx, ones_v, rows):
        c = lax.axis_index("c")
        s = lax.axis_index("s")
        wid = s * _NC + c
        base = wid * _EPT
        row0 = s * _RPT

        _zero_acc(zrows_hbm, rows, cnt_sh, row0)
        pltpu.sync_copy(ones_hbm, ones_v)
        plsc.subcore_barrier()

        def chunkc(i, carry):
            pltpu.sync_copy(dst_hbm.at[pl.ds(base + i * _CHC, _CHC)], didx)
            pltpu.sync_copy(ones_v, cnt_sh.at[didx], add=True)
            return carry

        lax.fori_loop(0, _EPT // _CHC, chunkc, 0)
        plsc.subcore_barrier()
        _dump_acc(cnt_sh, rows, cnt_out, row0, c)

    return cntk


_seg_sum = _make_seg_sum()
_cnt_scatter = _make_cnt()


def kernel(x, edge_index, num_nodes_type_1, num_nodes_type_2,
           W_src, b_src, W_dst, b_dst,
           Wl0, bl0, Wr0, gamma0, beta0, Wl1, bl1, Wr1):
    del num_nodes_type_1, num_nodes_type_2
    src = edge_index[0]
    dst = edge_index[1]
    npad = _EPAD - _E
    src_p = jnp.concatenate([src, jnp.zeros((npad + _CH,), jnp.int32)])
    dst_p = jnp.concatenate([dst, jnp.full((npad + _CH,), _N, jnp.int32)])
    zrows = jnp.zeros((_BB, _D), F32)
    ones_r = jnp.ones((_CHC, _D), F32)
    pad_i = jnp.full((_CH,), _N, jnp.int32)

    bs = b_src.reshape(1, _D)
    bd = b_dst.reshape(1, _D)
    bl0r = bl0.reshape(1, _D)
    bl1r = bl1.reshape(1, _D)
    g0 = gamma0.reshape(1, _D)
    be0 = beta0.reshape(1, _D)

    h = _proj(x, W_src, bs, W_dst, bd)
    cnt = _cnt_scatter(dst_p, zrows, ones_r).reshape(2, _NPAD, _D)
    recip = _cnt_recip(cnt)
    acc0 = _seg_sum(h, src_p, dst_p, zrows, pad_i).reshape(2, _NPAD, _D)
    hp0, s1, s2 = _sage_lin(acc0, recip, h, Wl0, bl0r, Wr0, with_stats=True)
    h2 = _bn_relu(hp0, s1, s2, g0, be0)
    acc1 = _seg_sum(h2, src_p, dst_p, zrows, pad_i).reshape(2, _NPAD, _D)
    return _sage_lin(acc1, recip, h2, Wl1, bl1r, Wr1, with_stats=False)[0]
